# baseline (speedup 1.0000x reference)
import functools
import math

import jax
import jax.numpy as jnp
from jax import lax
from jax.experimental import pallas as pl
from jax.experimental.pallas import tpu as pltpu
from jax.experimental.pallas import tpu_sc as plsc

_TEMP_INV = 1.0 / math.e
_NE = 8
_NA = 5
_CHUNKS = 4


def _scores_body(x_ref, w_ref, b_ref, o_ref):
    s = lax.dot_general(
        w_ref[...], x_ref[...], (((1,), (1,)), ((), ())),
        preferred_element_type=jnp.float32)
    o_ref[...] = (s + b_ref[...]) * _TEMP_INV


def _scores_tc(x, W, b2, chunk_idx, chunk_rows):
    d = x.shape[1]
    rows = min(4096, chunk_rows)
    blocks_per_chunk = chunk_rows // rows
    base = chunk_idx * blocks_per_chunk
    return pl.pallas_call(
        _scores_body,
        grid=(blocks_per_chunk,),
        in_specs=[
            pl.BlockSpec((rows, d), lambda i: (base + i, 0)),
            pl.BlockSpec((_NE, d), lambda i: (0, 0)),
            pl.BlockSpec((_NE, 1), lambda i: (0, 0)),
        ],
        out_specs=pl.BlockSpec((_NE, rows), lambda i: (0, i)),
        out_shape=jax.ShapeDtypeStruct((_NE, chunk_rows), jnp.float32),
    )(x, W, b2)


def _gate_sc(s_t):
    ne, n = s_t.shape
    info = plsc.get_sparse_core_info()
    nw = info.num_cores * info.num_subcores
    lanes = info.num_lanes
    chunk = n // nw
    mesh = plsc.VectorSubcoreMesh(core_axis_name="c", subcore_axis_name="s")

    @functools.partial(
        pl.kernel,
        out_type=jax.ShapeDtypeStruct((ne, n), jnp.float32),
        mesh=mesh,
        scratch_types=[
            pltpu.VMEM((ne, chunk), jnp.float32),
            pltpu.VMEM((ne, chunk), jnp.float32),
        ],
    )
    def gate(s_hbm, o_hbm, s_v, o_v):
        wid = lax.axis_index("s") * info.num_cores + lax.axis_index("c")
        base = wid * chunk
        pltpu.sync_copy(s_hbm.at[:, pl.ds(base, chunk)], s_v)

        def step(g, carry):
            col = g * lanes
            sv = [s_v[e, pl.ds(col, lanes)] for e in range(_NE)]
            m = sv[0]
            for e in range(1, _NE):
                m = jnp.maximum(m, sv[e])
            ev = [jnp.exp(v - m) for v in sv]
            z = ev[0]
            for e in range(1, _NE):
                z = z + ev[e]
            pv = [v / z for v in ev]

            one = jnp.ones((lanes,), jnp.float32)
            zero = jnp.zeros((lanes,), jnp.float32)
            kept = []
            for i in range(_NE):
                rank = zero
                for j in range(_NE):
                    if j < i:
                        hit = pv[j] >= pv[i]
                    elif j > i:
                        hit = pv[j] > pv[i]
                    else:
                        continue
                    rank = rank + jnp.where(hit, one, zero)
                keep = rank < (_NA - 0.5)
                kept.append(jnp.where(keep, pv[i], zero))
            denom = kept[0]
            for e in range(1, _NE):
                denom = denom + kept[e]
            denom = denom + 1e-8
            for e in range(_NE):
                o_v[e, pl.ds(col, lanes)] = kept[e] / denom
            return carry

        lax.fori_loop(0, chunk // lanes, step, 0)
        pltpu.sync_copy(o_v, o_hbm.at[:, pl.ds(base, chunk)])

    return gate(s_t)


def kernel(x, W, b):
    n = x.shape[0]
    b2 = b.reshape(_NE, 1)
    chunk_rows = n // _CHUNKS
    outs = []
    for k in range(_CHUNKS):
        s_k = _scores_tc(x, W, b2, k, chunk_rows)
        outs.append(_gate_sc(s_k))
    return jnp.concatenate(outs, axis=1).T


# speedup vs baseline: 1.1599x; 1.1599x over previous
import functools
import math

import jax
import jax.numpy as jnp
from jax import lax
from jax.experimental import pallas as pl
from jax.experimental.pallas import tpu as pltpu
from jax.experimental.pallas import tpu_sc as plsc

_TEMP_INV = 1.0 / math.e
_NE = 8
_NA = 5


def _scores_body(x_ref, w_ref, b_ref, o_ref):
    s = lax.dot_general(
        w_ref[...], x_ref[...], (((1,), (1,)), ((), ())),
        preferred_element_type=jnp.float32)
    o_ref[...] = (s + b_ref[...]) * _TEMP_INV


def _scores_tc(x, W, b2):
    n, d = x.shape
    rows = 4096
    return pl.pallas_call(
        _scores_body,
        grid=(n // rows,),
        in_specs=[
            pl.BlockSpec((rows, d), lambda i: (i, 0)),
            pl.BlockSpec((_NE, d), lambda i: (0, 0)),
            pl.BlockSpec((_NE, 1), lambda i: (0, 0)),
        ],
        out_specs=pl.BlockSpec((_NE, rows), lambda i: (0, i)),
        out_shape=jax.ShapeDtypeStruct((_NE, n), jnp.float32),
    )(x, W, b2)


def _gate_sc(s_t):
    ne, n = s_t.shape
    info = plsc.get_sparse_core_info()
    nw = info.num_cores * info.num_subcores
    lanes = info.num_lanes
    chunk = n // nw
    mesh = plsc.VectorSubcoreMesh(core_axis_name="c", subcore_axis_name="s")

    @functools.partial(
        pl.kernel,
        out_type=jax.ShapeDtypeStruct((ne, n), jnp.float32),
        mesh=mesh,
        scratch_types=[
            pltpu.VMEM((ne, chunk), jnp.float32),
            pltpu.VMEM((ne, chunk), jnp.float32),
        ],
    )
    def gate(s_hbm, o_hbm, s_v, o_v):
        wid = lax.axis_index("s") * info.num_cores + lax.axis_index("c")
        base = wid * chunk
        pltpu.sync_copy(s_hbm.at[:, pl.ds(base, chunk)], s_v)
        zero = jnp.zeros((lanes,), jnp.float32)
        one = jnp.ones((lanes,), jnp.float32)

        def gate_group(col):
            sv = [s_v[e, pl.ds(col, lanes)] for e in range(_NE)]
            m = sv[0]
            for e in range(1, _NE):
                m = jnp.maximum(m, sv[e])
            ev = [jnp.exp(v - m) for v in sv]
            z = ev[0]
            for e in range(1, _NE):
                z = z + ev[e]
            kept = []
            for i in range(_NE):
                rank = zero
                for j in range(_NE):
                    if j < i:
                        hit = ev[j] >= ev[i]
                    elif j > i:
                        hit = ev[j] > ev[i]
                    else:
                        continue
                    rank = rank + jnp.where(hit, one, zero)
                keep = rank < (_NA - 0.5)
                kept.append(jnp.where(keep, ev[i], zero))
            s_kept = kept[0]
            for e in range(1, _NE):
                s_kept = s_kept + kept[e]
            inv = 1.0 / (s_kept + 1e-8 * z)
            for e in range(_NE):
                o_v[e, pl.ds(col, lanes)] = kept[e] * inv

        def step(g, carry):
            gate_group(g * 2 * lanes)
            gate_group(g * 2 * lanes + lanes)
            return carry

        lax.fori_loop(0, chunk // (2 * lanes), step, 0)
        pltpu.sync_copy(o_v, o_hbm.at[:, pl.ds(base, chunk)])

    return gate(s_t)


def kernel(x, W, b):
    s_t = _scores_tc(x, W, b.reshape(_NE, 1))
    return _gate_sc(s_t).T
